# baseline (speedup 1.0000x reference)
import functools

import jax
import jax.numpy as jnp
from jax import lax
from jax.experimental import pallas as pl
from jax.experimental.pallas import tpu as pltpu
from jax.experimental.pallas import tpu_sc as plsc

GX, GY, GZ = 512, 512, 10
VX, VY, VZ = 0.2, 0.2, 0.8
XMIN, YMIN, ZMIN = -51.2, -51.2, -5.0
K = 40000
NP = 300000
NV = GX * GY * GZ
NPP = 300032
CAP = 32
ROWS = NV // 128
CHUNK_ROWS = 16
NCHUNK = ROWS // CHUNK_ROWS
PROG_ROWS = 1024
NPROG = ROWS // PROG_ROWS
CPP = PROG_ROWS // CHUNK_ROWS
KPAD = 40960
KVOX = 40968
KACC = 41088

NSC = 2
NSUB = 16
NW = NSC * NSUB
PPT = NPP // NW
PPS = NPP // NSUB
HALF = NV // 2
HTAB = HALF + 8
HSTR = HALF // NSUB
ASTR = KACC // NSUB
_SC_MESH = plsc.VectorSubcoreMesh(core_axis_name="c", subcore_axis_name="s")


K1_PASS = 4
K1_CH = PPS // K1_PASS


def _sc_counts_body(ids_hbm, ones_hbm, zeros_hbm, out_hbm,
                    ids_v, idx_v, ones_v, shared, sem):
    c = lax.axis_index("c")
    s = lax.axis_index("s")
    pltpu.sync_copy(zeros_hbm, shared.at[pl.ds(s * HSTR, HSTR)])
    pltpu.sync_copy(ones_hbm, ones_v)
    lo = c * HALF
    plsc.subcore_barrier()
    for p in range(K1_PASS):
        pltpu.sync_copy(ids_hbm.at[pl.ds(s * PPS + p * K1_CH, K1_CH)], ids_v)

        def body(i, _):
            idsv = ids_v[pl.ds(i * 16, 16)]
            inhalf = (idsv >= lo) & (idsv < lo + HALF)
            idx_v[pl.ds(i * 16, 16)] = jnp.where(inhalf, idsv - lo, HALF)
            return 0

        lax.fori_loop(0, K1_CH // 16, body, 0)
        pltpu.sync_copy(ones_v, shared.at[idx_v], add=True)
    plsc.subcore_barrier()
    pltpu.sync_copy(shared.at[pl.ds(s * HSTR, HSTR)],
                    out_hbm.at[pl.ds(c * HALF + s * HSTR, HSTR)])


@functools.partial(
    pl.kernel, mesh=_SC_MESH,
    out_type=jax.ShapeDtypeStruct((NV,), jnp.float32),
    scratch_types=[
        pltpu.VMEM((K1_CH,), jnp.int32),
        pltpu.VMEM((K1_CH,), jnp.int32),
        pltpu.VMEM((K1_CH,), jnp.float32),
        pltpu.VMEM_SHARED((HTAB,), jnp.float32),
        pltpu.SemaphoreType.DMA,
    ],
)
def _sc_counts(*args):
    _sc_counts_body(*args)


K5_PASS = 4
K5_CH = NV // NW // K5_PASS


def _sc_slots_body(pos_hbm, out_hbm, pos_v, val_v, sem):
    c = lax.axis_index("c")
    s = lax.axis_index("s")
    w = s * NSC + c
    for p in range(K5_PASS):
        base = w * (NV // NW) + p * K5_CH
        pltpu.sync_copy(pos_hbm.at[pl.ds(base, K5_CH)], pos_v)

        def body(j, _):
            val_v[pl.ds(j * 16, 16)] = base + j * 16 + lax.iota(jnp.int32, 16)
            return 0

        lax.fori_loop(0, K5_CH // 16, body, 0)
        pltpu.async_copy(val_v, out_hbm.at[pos_v], sem).wait()


@functools.partial(
    pl.kernel, mesh=_SC_MESH,
    out_type=jax.ShapeDtypeStruct((KVOX,), jnp.int32),
    scratch_types=[
        pltpu.VMEM((K5_CH,), jnp.int32),
        pltpu.VMEM((K5_CH,), jnp.int32),
        pltpu.SemaphoreType.DMA,
    ],
)
def _sc_slots(*args):
    _sc_slots_body(*args)


def _sc_accum_body(featst_hbm, ids_hbm, tab_hbm, zeros_hbm, out_hbm,
                   ids_v, slots_v, feat_v, shared, sem):
    c = lax.axis_index("c")
    s = lax.axis_index("s")
    w = s * NSC + c
    for f in range(6):
        pltpu.sync_copy(zeros_hbm, shared.at[f, pl.ds(s * ASTR, ASTR)])
    pltpu.sync_copy(ids_hbm.at[pl.ds(w * PPT, PPT)], ids_v)
    pltpu.async_copy(tab_hbm.at[ids_v], slots_v, sem).wait()
    plsc.subcore_barrier()
    for f in range(6):
        pltpu.sync_copy(featst_hbm.at[f, pl.ds(w * PPT, PPT)], feat_v)
        pltpu.sync_copy(feat_v, shared.at[f].at[slots_v], add=True)
    plsc.subcore_barrier()
    for f in range(6):
        pltpu.sync_copy(shared.at[f, pl.ds(s * ASTR, ASTR)],
                        out_hbm.at[c, f, pl.ds(s * ASTR, ASTR)])


@functools.partial(
    pl.kernel, mesh=_SC_MESH,
    compiler_params=pltpu.CompilerParams(use_tc_tiling_on_sc=False),
    out_type=jax.ShapeDtypeStruct((NSC, 6, KACC), jnp.float32),
    scratch_types=[
        pltpu.VMEM((PPT,), jnp.int32),
        pltpu.VMEM((PPT,), jnp.int32),
        pltpu.VMEM((PPT,), jnp.float32),
        pltpu.VMEM_SHARED((6, KACC), jnp.float32),
        pltpu.SemaphoreType.DMA,
    ],
)
def _sc_accum(*args):
    _sc_accum_body(*args)


def _k0_body(pts_ref, out_ref):
    i = pl.program_id(0)
    x = pts_ref[0:1, :]
    y = pts_ref[1:2, :]
    z = pts_ref[2:3, :]
    cx = jnp.floor((x - XMIN) / VX).astype(jnp.int32)
    cy = jnp.floor((y - YMIN) / VY).astype(jnp.int32)
    cz = jnp.floor((z - ZMIN) / VZ).astype(jnp.int32)
    valid = ((cx >= 0) & (cx < GX) & (cy >= 0) & (cy < GY)
             & (cz >= 0) & (cz < GZ))
    gidx = i * (NPP // 8) + jax.lax.broadcasted_iota(jnp.int32, x.shape, 1)
    valid = valid & (gidx < NP)
    flat = cz * (GX * GY) + cy * GX + cx
    out_ref[...] = jnp.where(valid, flat, NV)


def _flat_ids(points):
    pts_pad = jnp.zeros((NPP, 5), jnp.float32).at[:NP].set(points)
    pts_t = pts_pad.T
    ids2d = pl.pallas_call(
        _k0_body,
        grid=(8,),
        in_specs=[pl.BlockSpec((5, NPP // 8), lambda i: (0, i))],
        out_specs=pl.BlockSpec((1, NPP // 8), lambda i: (0, i)),
        out_shape=jax.ShapeDtypeStruct((1, NPP), jnp.int32),
    )(pts_t)
    return ids2d.reshape(NPP)


def _k2_body(cnt_ref, hist_ref):
    v = jnp.minimum(cnt_ref[...].astype(jnp.int32), CAP - 1)
    cols = [jnp.sum((v == b).astype(jnp.float32), axis=1, keepdims=True)
            for b in range(CAP)]
    h = jnp.concatenate(cols, axis=1)
    r = jax.lax.broadcasted_iota(jnp.int32, (CPP, PROG_ROWS), 1)
    g = jax.lax.broadcasted_iota(jnp.int32, (CPP, PROG_ROWS), 0)
    sel = (r // CHUNK_ROWS == g).astype(jnp.float32)
    hist_ref[...] = jnp.dot(sel, h, preferred_element_type=jnp.float32,
                 precision=jax.lax.Precision.HIGHEST)


def _k3_body(hist_ref, comb_ref):
    h = hist_ref[...]
    i0 = jax.lax.broadcasted_iota(jnp.int32, (NCHUNK, NCHUNK), 0)
    i1 = jax.lax.broadcasted_iota(jnp.int32, (NCHUNK, NCHUNK), 1)
    lower = (i1 < i0).astype(jnp.float32)
    excl = jnp.dot(lower, h, preferred_element_type=jnp.float32,
                 precision=jax.lax.Precision.HIGHEST)
    total = jnp.sum(h, axis=0, keepdims=True)
    b0 = jax.lax.broadcasted_iota(jnp.int32, (CAP, CAP), 0)
    b1 = jax.lax.broadcasted_iota(jnp.int32, (CAP, CAP), 1)
    gt = (b0 > b1).astype(jnp.float32)
    ng = jnp.dot(total, gt, preferred_element_type=jnp.float32,
                 precision=jax.lax.Precision.HIGHEST)
    comb_ref[...] = excl + ng


def _k4_body(cnt_ref, comb_ref, pos_ref):
    v = jnp.minimum(cnt_ref[...].astype(jnp.int32), CAP - 1)
    cols = [jnp.sum((v == b).astype(jnp.float32), axis=1, keepdims=True)
            for b in range(CAP)]
    h = jnp.concatenate(cols, axis=1)
    r0 = jax.lax.broadcasted_iota(jnp.int32, (PROG_ROWS, PROG_ROWS), 0)
    r1 = jax.lax.broadcasted_iota(jnp.int32, (PROG_ROWS, PROG_ROWS), 1)
    bd = ((r0 // CHUNK_ROWS == r1 // CHUNK_ROWS)
          & (r1 < r0)).astype(jnp.float32)
    base32 = jnp.dot(bd, h, preferred_element_type=jnp.float32,
                 precision=jax.lax.Precision.HIGHEST)
    rr = jax.lax.broadcasted_iota(jnp.int32, (PROG_ROWS, CPP), 0)
    gg = jax.lax.broadcasted_iota(jnp.int32, (PROG_ROWS, CPP), 1)
    rep = (rr // CHUNK_ROWS == gg).astype(jnp.float32)
    base32 = base32 + jnp.dot(rep, comb_ref[...],
                              preferred_element_type=jnp.float32,
                 precision=jax.lax.Precision.HIGHEST)
    l0 = jax.lax.broadcasted_iota(jnp.int32, (128, 128), 0)
    l1 = jax.lax.broadcasted_iota(jnp.int32, (128, 128), 1)
    u = (l0 < l1).astype(jnp.float32)
    posf = jnp.zeros(v.shape, jnp.float32)
    for b in range(CAP):
        eqb = (v == b).astype(jnp.float32)
        lane_excl = jnp.dot(eqb, u, preferred_element_type=jnp.float32,
                 precision=jax.lax.Precision.HIGHEST)
        posf = posf + eqb * (lane_excl + base32[:, b:b + 1])
    pos_ref[...] = jnp.minimum(posf, float(KPAD)).astype(jnp.int32)


def _positions(counts2d):
    hists = pl.pallas_call(
        _k2_body,
        grid=(NPROG,),
        in_specs=[pl.BlockSpec((PROG_ROWS, 128), lambda i: (i, 0))],
        out_specs=pl.BlockSpec((CPP, CAP), lambda i: (i, 0)),
        out_shape=jax.ShapeDtypeStruct((NCHUNK, CAP), jnp.float32),
    )(counts2d)
    comb = pl.pallas_call(
        _k3_body,
        out_shape=jax.ShapeDtypeStruct((NCHUNK, CAP), jnp.float32),
    )(hists)
    pos2d = pl.pallas_call(
        _k4_body,
        grid=(NPROG,),
        in_specs=[pl.BlockSpec((PROG_ROWS, 128), lambda i: (i, 0)),
                  pl.BlockSpec((CPP, CAP), lambda i: (i, 0))],
        out_specs=pl.BlockSpec((PROG_ROWS, 128), lambda i: (i, 0)),
        out_shape=jax.ShapeDtypeStruct((ROWS, 128), jnp.int32),
    )(counts2d, comb)
    return pos2d


def _k7_body(a0_ref, a1_ref, vox_ref, out_ref):
    s = a0_ref[...] + a1_ref[...]
    cnt = s[5:6, :]
    feats = s[0:5, :] / jnp.maximum(cnt, 1.0)
    feats = feats * (cnt > 0).astype(jnp.float32)
    vox = vox_ref[0:1, :]
    zc = vox // (GX * GY)
    rem = vox - zc * (GX * GY)
    yc = rem // GX
    xc = rem - yc * GX
    coords = jnp.concatenate([zc, yc, xc], axis=0).astype(jnp.float32)
    pad = jnp.zeros((7, s.shape[1]), jnp.float32)
    out_ref[...] = jnp.concatenate([feats, cnt, coords, pad], axis=0)


def _finalize(a0t, a1t, voxid):
    voxb = jnp.broadcast_to(voxid[None, :], (8, KPAD))
    out = pl.pallas_call(
        _k7_body,
        out_shape=jax.ShapeDtypeStruct((16, KPAD), jnp.float32),
    )(a0t, a1t, voxb)
    feats = out[0:5, :K].T
    cnts = out[5, :K].astype(jnp.int32)
    coords = out[6:9, :K].T.astype(jnp.int32)
    return feats, coords, cnts


def kernel(points):
    ids_pad = _flat_ids(points)

    counts = _sc_counts(ids_pad,
                        jnp.ones((K1_CH,), jnp.float32),
                        jnp.zeros((HSTR,), jnp.float32))
    counts2d = counts.reshape(ROWS, 128)

    pos2d = _positions(counts2d)
    pos = pos2d.reshape(NV)

    voxid = _sc_slots(pos)[:KPAD]

    slot_tab = jnp.concatenate([pos, jnp.full((8,), KPAD, jnp.int32)])
    featst = (jnp.zeros((6, NPP), jnp.float32)
              .at[:5, :NP].set(points.T).at[5, :NP].set(1.0))
    accum = _sc_accum(featst, ids_pad, slot_tab,
                      jnp.zeros((ASTR,), jnp.float32))
    a0t = accum[0, :, :KPAD]
    a1t = accum[1, :, :KPAD]

    return _finalize(a0t, a1t, voxid)


# speedup vs baseline: 211.0375x; 211.0375x over previous
import functools

import jax
import jax.numpy as jnp
from jax import lax
from jax.experimental import pallas as pl
from jax.experimental.pallas import tpu as pltpu
from jax.experimental.pallas import tpu_sc as plsc

GX, GY, GZ = 512, 512, 10
VX, VY, VZ = 0.2, 0.2, 0.8
XMIN, YMIN, ZMIN = -51.2, -51.2, -5.0
K = 40000
NP = 300000
NV = GX * GY * GZ
NPP = 300032
CAP = 32
ROWS = NV // 128
CHUNK_ROWS = 16
NCHUNK = ROWS // CHUNK_ROWS
PROG_ROWS = 1024
NPROG = ROWS // PROG_ROWS
CPP = PROG_ROWS // CHUNK_ROWS
KPAD = 40960
KVOX = 40968
KACC = 41088

NSC = 2
NSUB = 16
NW = NSC * NSUB
PPT = NPP // NW
PPS = NPP // NSUB
HALF = NV // 2
HTAB = HALF + 8
HSTR = HALF // NSUB
ASTR = KACC // NSUB
_SC_MESH = plsc.VectorSubcoreMesh(core_axis_name="c", subcore_axis_name="s")


K1_PASS = 4
K1_CH = PPS // K1_PASS


def _sc_counts_body(ids_hbm, ones_hbm, zeros_hbm, out_hbm,
                    ids_v, idx_v, ones_v, shared, sem):
    c = lax.axis_index("c")
    s = lax.axis_index("s")
    pltpu.sync_copy(zeros_hbm, shared.at[pl.ds(s * HSTR, HSTR)])
    pltpu.sync_copy(ones_hbm, ones_v)
    lo = c * HALF
    plsc.subcore_barrier()
    for p in range(K1_PASS):
        pltpu.sync_copy(ids_hbm.at[pl.ds(s * PPS + p * K1_CH, K1_CH)], ids_v)

        def body(i, _):
            idsv = ids_v[pl.ds(i * 16, 16)]
            inhalf = (idsv >= lo) & (idsv < lo + HALF)
            idx_v[pl.ds(i * 16, 16)] = jnp.where(inhalf, idsv - lo, HALF)
            return 0

        lax.fori_loop(0, K1_CH // 16, body, 0)
        pltpu.sync_copy(ones_v, shared.at[idx_v], add=True)
    plsc.subcore_barrier()
    pltpu.sync_copy(shared.at[pl.ds(s * HSTR, HSTR)],
                    out_hbm.at[pl.ds(c * HALF + s * HSTR, HSTR)])


@functools.partial(
    pl.kernel, mesh=_SC_MESH,
    out_type=jax.ShapeDtypeStruct((NV,), jnp.float32),
    scratch_types=[
        pltpu.VMEM((K1_CH,), jnp.int32),
        pltpu.VMEM((K1_CH,), jnp.int32),
        pltpu.VMEM((K1_CH,), jnp.float32),
        pltpu.VMEM_SHARED((HTAB,), jnp.float32),
        pltpu.SemaphoreType.DMA,
    ],
)
def _sc_counts(*args):
    _sc_counts_body(*args)


def _sc_accum_body(featst_hbm, ids_hbm, tab_hbm, zeros_hbm, out_hbm,
                   ids_v, slots_v, feat_v, shared, sem):
    c = lax.axis_index("c")
    s = lax.axis_index("s")
    w = s * NSC + c
    for f in range(8):
        pltpu.sync_copy(zeros_hbm, shared.at[f, pl.ds(s * ASTR, ASTR)])
    pltpu.sync_copy(ids_hbm.at[pl.ds(w * PPT, PPT)], ids_v)
    pltpu.async_copy(tab_hbm.at[ids_v], slots_v, sem).wait()
    plsc.subcore_barrier()
    for f in range(8):
        pltpu.sync_copy(featst_hbm.at[f, pl.ds(w * PPT, PPT)], feat_v)
        pltpu.sync_copy(feat_v, shared.at[f].at[slots_v], add=True)
    plsc.subcore_barrier()
    for f in range(8):
        pltpu.sync_copy(shared.at[f, pl.ds(s * ASTR, ASTR)],
                        out_hbm.at[c, f, pl.ds(s * ASTR, ASTR)])


@functools.partial(
    pl.kernel, mesh=_SC_MESH,
    compiler_params=pltpu.CompilerParams(use_tc_tiling_on_sc=False),
    out_type=jax.ShapeDtypeStruct((NSC, 8, KACC), jnp.float32),
    scratch_types=[
        pltpu.VMEM((PPT,), jnp.int32),
        pltpu.VMEM((PPT,), jnp.int32),
        pltpu.VMEM((PPT,), jnp.float32),
        pltpu.VMEM_SHARED((8, KACC), jnp.float32),
        pltpu.SemaphoreType.DMA,
    ],
)
def _sc_accum(*args):
    _sc_accum_body(*args)


def _k0_body(pts_ref, out_ref, hi_ref, lo_ref):
    i = pl.program_id(0)
    x = pts_ref[0:1, :]
    y = pts_ref[1:2, :]
    z = pts_ref[2:3, :]
    cx = jnp.floor((x - XMIN) / VX).astype(jnp.int32)
    cy = jnp.floor((y - YMIN) / VY).astype(jnp.int32)
    cz = jnp.floor((z - ZMIN) / VZ).astype(jnp.int32)
    valid = ((cx >= 0) & (cx < GX) & (cy >= 0) & (cy < GY)
             & (cz >= 0) & (cz < GZ))
    gidx = i * (NPP // 8) + jax.lax.broadcasted_iota(jnp.int32, x.shape, 1)
    valid = valid & (gidx < NP)
    flat = jnp.where(valid, cz * (GX * GY) + cy * GX + cx, NV)
    out_ref[...] = flat
    hi_ref[...] = (flat // 2048).astype(jnp.float32)
    lo_ref[...] = (flat % 2048).astype(jnp.float32)


def _flat_ids(points):
    pts_pad = jnp.zeros((NPP, 5), jnp.float32).at[:NP].set(points)
    pts_t = pts_pad.T
    ids2d = pl.pallas_call(
        _k0_body,
        grid=(8,),
        in_specs=[pl.BlockSpec((5, NPP // 8), lambda i: (0, i))],
        out_specs=[pl.BlockSpec((1, NPP // 8), lambda i: (0, i))] * 3,
        out_shape=[jax.ShapeDtypeStruct((1, NPP), jnp.int32),
                   jax.ShapeDtypeStruct((1, NPP), jnp.float32),
                   jax.ShapeDtypeStruct((1, NPP), jnp.float32)],
    )(pts_t)
    ids2d, hi2d, lo2d = ids2d
    return ids2d.reshape(NPP), hi2d, lo2d


def _k2_body(cnt_ref, hist_ref):
    v = jnp.minimum(cnt_ref[...].astype(jnp.int32), CAP - 1)
    cols = [jnp.sum((v == b).astype(jnp.float32), axis=1, keepdims=True)
            for b in range(CAP)]
    h = jnp.concatenate(cols, axis=1)
    r = jax.lax.broadcasted_iota(jnp.int32, (CPP, PROG_ROWS), 1)
    g = jax.lax.broadcasted_iota(jnp.int32, (CPP, PROG_ROWS), 0)
    sel = (r // CHUNK_ROWS == g).astype(jnp.float32)
    hist_ref[...] = jnp.dot(sel, h, preferred_element_type=jnp.float32,
                 precision=jax.lax.Precision.HIGHEST)


def _k3_body(hist_ref, comb_ref):
    h = hist_ref[...]
    i0 = jax.lax.broadcasted_iota(jnp.int32, (NCHUNK, NCHUNK), 0)
    i1 = jax.lax.broadcasted_iota(jnp.int32, (NCHUNK, NCHUNK), 1)
    lower = (i1 < i0).astype(jnp.float32)
    excl = jnp.dot(lower, h, preferred_element_type=jnp.float32,
                 precision=jax.lax.Precision.HIGHEST)
    total = jnp.sum(h, axis=0, keepdims=True)
    b0 = jax.lax.broadcasted_iota(jnp.int32, (CAP, CAP), 0)
    b1 = jax.lax.broadcasted_iota(jnp.int32, (CAP, CAP), 1)
    gt = (b0 > b1).astype(jnp.float32)
    ng = jnp.dot(total, gt, preferred_element_type=jnp.float32,
                 precision=jax.lax.Precision.HIGHEST)
    comb_ref[...] = excl + ng


def _k4_body(cnt_ref, comb_ref, pos_ref):
    v = jnp.minimum(cnt_ref[...].astype(jnp.int32), CAP - 1)
    cols = [jnp.sum((v == b).astype(jnp.float32), axis=1, keepdims=True)
            for b in range(CAP)]
    h = jnp.concatenate(cols, axis=1)
    r0 = jax.lax.broadcasted_iota(jnp.int32, (PROG_ROWS, PROG_ROWS), 0)
    r1 = jax.lax.broadcasted_iota(jnp.int32, (PROG_ROWS, PROG_ROWS), 1)
    bd = ((r0 // CHUNK_ROWS == r1 // CHUNK_ROWS)
          & (r1 < r0)).astype(jnp.float32)
    base32 = jnp.dot(bd, h, preferred_element_type=jnp.float32,
                 precision=jax.lax.Precision.HIGHEST)
    rr = jax.lax.broadcasted_iota(jnp.int32, (PROG_ROWS, CPP), 0)
    gg = jax.lax.broadcasted_iota(jnp.int32, (PROG_ROWS, CPP), 1)
    rep = (rr // CHUNK_ROWS == gg).astype(jnp.float32)
    base32 = base32 + jnp.dot(rep, comb_ref[...],
                              preferred_element_type=jnp.float32,
                 precision=jax.lax.Precision.HIGHEST)
    l0 = jax.lax.broadcasted_iota(jnp.int32, (128, 128), 0)
    l1 = jax.lax.broadcasted_iota(jnp.int32, (128, 128), 1)
    u = (l0 < l1).astype(jnp.float32)
    posf = jnp.zeros(v.shape, jnp.float32)
    for b in range(CAP):
        eqb = (v == b).astype(jnp.float32)
        lane_excl = jnp.dot(eqb, u, preferred_element_type=jnp.float32,
                 precision=jax.lax.Precision.HIGHEST)
        posf = posf + eqb * (lane_excl + base32[:, b:b + 1])
    pos_ref[...] = jnp.minimum(posf, float(KPAD)).astype(jnp.int32)


def _positions(counts2d):
    hists = pl.pallas_call(
        _k2_body,
        grid=(NPROG,),
        in_specs=[pl.BlockSpec((PROG_ROWS, 128), lambda i: (i, 0))],
        out_specs=pl.BlockSpec((CPP, CAP), lambda i: (i, 0)),
        out_shape=jax.ShapeDtypeStruct((NCHUNK, CAP), jnp.float32),
    )(counts2d)
    comb = pl.pallas_call(
        _k3_body,
        out_shape=jax.ShapeDtypeStruct((NCHUNK, CAP), jnp.float32),
    )(hists)
    pos2d = pl.pallas_call(
        _k4_body,
        grid=(NPROG,),
        in_specs=[pl.BlockSpec((PROG_ROWS, 128), lambda i: (i, 0)),
                  pl.BlockSpec((CPP, CAP), lambda i: (i, 0))],
        out_specs=pl.BlockSpec((PROG_ROWS, 128), lambda i: (i, 0)),
        out_shape=jax.ShapeDtypeStruct((ROWS, 128), jnp.int32),
    )(counts2d, comb)
    return pos2d


def _k7_body(a0_ref, a1_ref, out_ref):
    s = a0_ref[...] + a1_ref[...]
    cnt = s[5:6, :]
    safe = jnp.maximum(cnt, 1.0)
    feats = s[0:5, :] / safe
    feats = feats * (cnt > 0).astype(jnp.float32)
    vox = (s[6:7, :] / safe).astype(jnp.int32) * 2048 \
        + (s[7:8, :] / safe).astype(jnp.int32)
    zc = vox // (GX * GY)
    rem = vox - zc * (GX * GY)
    yc = rem // GX
    xc = rem - yc * GX
    coords = jnp.concatenate([zc, yc, xc], axis=0).astype(jnp.float32)
    pad = jnp.zeros((7, s.shape[1]), jnp.float32)
    out_ref[...] = jnp.concatenate([feats, cnt, coords, pad], axis=0)


def _finalize(a0t, a1t):
    out = pl.pallas_call(
        _k7_body,
        out_shape=jax.ShapeDtypeStruct((16, KPAD), jnp.float32),
    )(a0t, a1t)
    feats = out[0:5, :K].T
    cnts = out[5, :K].astype(jnp.int32)
    coords = out[6:9, :K].T.astype(jnp.int32)
    return feats, coords, cnts


def kernel(points):
    ids_pad, hi2d, lo2d = _flat_ids(points)

    counts = _sc_counts(ids_pad,
                        jnp.ones((K1_CH,), jnp.float32),
                        jnp.zeros((HSTR,), jnp.float32))
    counts2d = counts.reshape(ROWS, 128)

    pos2d = _positions(counts2d)
    pos = pos2d.reshape(NV)

    slot_tab = jnp.concatenate([pos, jnp.full((8,), KPAD, jnp.int32)])
    featst = jnp.concatenate(
        [jnp.zeros((5, NPP), jnp.float32).at[:, :NP].set(points.T),
         jnp.zeros((1, NPP), jnp.float32).at[:, :NP].set(1.0),
         hi2d, lo2d], axis=0)
    accum = _sc_accum(featst, ids_pad, slot_tab,
                      jnp.zeros((ASTR,), jnp.float32))
    a0t = accum[0, :, :KPAD]
    a1t = accum[1, :, :KPAD]

    return _finalize(a0t, a1t)
